# Initial kernel scaffold; baseline (speedup 1.0000x reference)
import jax
import jax.numpy as jnp
from jax.experimental import pallas as pl


def kernel(x, pos_edge_index, neg_edge_index, t, W_enc, b_enc, W_pos, b_pos, W_neg, b_neg, W_ode, b_ode):
    raise NotImplementedError("write your pallas kernel here")


# speedup vs baseline: 27.4269x; 27.4269x over previous
import functools

import jax
import jax.numpy as jnp
from jax import lax
from jax.experimental import pallas as pl
from jax.experimental.pallas import tpu as pltpu
from jax.experimental.pallas import tpu_sc as plsc

_N = 50000
_F = 128
_H = 64
_E = 800000
_STEPS = 8

_NS = 16
_NC = 2
_LANE = 128
_KG = 8
_KGS = 4
_GRPS = _KGS * _LANE
_NGRP = -(-_E // (_KG * _LANE * _NS))
_EPT = _NGRP * _KG * _LANE
_NGRPS = _EPT // _GRPS
_EP = _EPT * _NS
_EROWS = _EP // _LANE
_ERPT = _EPT // _LANE
_NPAD = 51200
_RPT = _NPAD // _NS
_RCHUNKS = ((0, 1024), (1024, 1024), (2048, 1024), (3072, 128))
_ZB = 1024
_SCHUNKS = ((0, 512), (512, 512), (1024, 512), (1536, 512),
            (2048, 512), (2560, 512), (3072, 128))

_A = (
    (),
    (1 / 5,),
    (3 / 40, 9 / 40),
    (44 / 45, -56 / 15, 32 / 9),
    (19372 / 6561, -25360 / 2187, 64448 / 6561, -212 / 729),
    (9017 / 3168, -355 / 33, 46732 / 5247, 49 / 176, -5103 / 18656),
)
_B = (35 / 384, 0.0, 500 / 1113, 125 / 192, -2187 / 6784, 11 / 84)

_f32 = jnp.float32


def _deg_body(dstall_hbm, deg_out_hbm, didx_v, ones_v, zb_v, deg_sh, sem):
    c = lax.axis_index("c")
    s = lax.axis_index("s")
    for j in range(_LANE // 16):
        ones_v[pl.ds(j * 16, 16)] = jnp.ones((16,), _f32)

    def _zero(i, carry):
        zb_v[pl.ds(i * 16, 16)] = jnp.zeros((16,), _f32)
        return carry

    lax.fori_loop(0, _ZB // 16, _zero, 0)
    base = s * _RPT
    for off, sz in _RCHUNKS:
        pltpu.sync_copy(zb_v.at[pl.ds(0, sz)], deg_sh.at[pl.ds(base + off, sz)])
    plsc.subcore_barrier()

    def _group(g, carry):
        rowbase = s * _ERPT + g * _KG
        pltpu.sync_copy(dstall_hbm.at[c].at[pl.ds(rowbase, _KG)], didx_v)
        descs = [
            pltpu.async_copy(ones_v, deg_sh.at[didx_v.at[j]], sem, add=True)
            for j in range(_KG)
        ]
        for d in descs:
            d.wait()
        return carry

    lax.fori_loop(0, _NGRP, _group, 0)
    plsc.subcore_barrier()
    for off, sz in _RCHUNKS:
        pltpu.sync_copy(deg_sh.at[pl.ds(base + off, sz)], zb_v.at[pl.ds(0, sz)])
        pltpu.sync_copy(zb_v.at[pl.ds(0, sz)],
                        deg_out_hbm.at[c].at[pl.ds(base + off, sz)])


@functools.lru_cache(maxsize=None)
def _deg_kernel():
    return pl.kernel(
        _deg_body,
        out_type=jax.ShapeDtypeStruct((_NC, _NPAD), _f32),
        mesh=plsc.VectorSubcoreMesh(core_axis_name="c", subcore_axis_name="s"),
        scratch_types=[
            pltpu.VMEM((_KG, _LANE), jnp.int32),
            pltpu.VMEM((_LANE,), _f32),
            pltpu.VMEM((_ZB,), _f32),
            pltpu.VMEM_SHARED((_NPAD,), _f32),
            pltpu.SemaphoreType.DMA,
        ],
        compiler_params=pltpu.CompilerParams(use_tc_tiling_on_sc=False),
    )


def _scatter_body(sall_hbm, srcall_hbm, dst_hbm, acc_out_hbm,
                  sidx_v, didx_v, rows_v, acc_sh, gsem, ssem):
    c = lax.axis_index("c")
    s = lax.axis_index("s")

    def _zero(i, carry):
        rows_v[i, pl.ds(0, 16)] = jnp.zeros((16,), _f32)
        rows_v[i, pl.ds(16, 16)] = jnp.zeros((16,), _f32)
        return carry

    lax.fori_loop(0, _GRPS, _zero, 0)
    base = s * _RPT
    for off, sz in _SCHUNKS:
        pltpu.sync_copy(rows_v.at[pl.ds(0, sz)], acc_sh.at[pl.ds(base + off, sz)])
    plsc.subcore_barrier()

    def _group(g, carry):
        rowbase = s * _ERPT + g * _KGS
        pltpu.sync_copy(srcall_hbm.at[c].at[pl.ds(rowbase, _KGS)], sidx_v)
        pltpu.sync_copy(dst_hbm.at[pl.ds(rowbase, _KGS)], didx_v)
        gd = [
            pltpu.async_copy(sall_hbm.at[sidx_v.at[j]],
                             rows_v.at[pl.ds(j * _LANE, _LANE)], gsem)
            for j in range(_KGS)
        ]
        sd = []
        for j in range(_KGS):
            gd[j].wait()
            sd.append(pltpu.async_copy(rows_v.at[pl.ds(j * _LANE, _LANE)],
                                       acc_sh.at[didx_v.at[j]], ssem, add=True))
        for d in sd:
            d.wait()
        return carry

    lax.fori_loop(0, _NGRPS, _group, 0)
    plsc.subcore_barrier()
    for off, sz in _SCHUNKS:
        pltpu.sync_copy(acc_sh.at[pl.ds(base + off, sz)], rows_v.at[pl.ds(0, sz)])
        pltpu.sync_copy(rows_v.at[pl.ds(0, sz)],
                        acc_out_hbm.at[c].at[pl.ds(base + off, sz)])


@functools.lru_cache(maxsize=None)
def _scatter_kernel():
    return pl.kernel(
        _scatter_body,
        out_type=jax.ShapeDtypeStruct((_NC, _NPAD, 32), _f32),
        mesh=plsc.VectorSubcoreMesh(core_axis_name="c", subcore_axis_name="s"),
        scratch_types=[
            pltpu.VMEM((_KGS, _LANE), jnp.int32),
            pltpu.VMEM((_KGS, _LANE), jnp.int32),
            pltpu.VMEM((_GRPS, 32), _f32),
            pltpu.VMEM_SHARED((_NPAD, 32), _f32),
            pltpu.SemaphoreType.DMA,
            pltpu.SemaphoreType.DMA,
        ],
        compiler_params=pltpu.CompilerParams(use_tc_tiling_on_sc=False),
    )


_BN = 1000


def _pre_body(x_ref, we_ref, be_ref, wp_ref, bp_ref, wn_ref, bn_ref, deg_ref,
              sp_ref, sn_ref, selfp_ref, selfn_ref, dinvp_ref, dinvn_ref):
    h = jnp.dot(x_ref[...], we_ref[...], preferred_element_type=_f32) + be_ref[...]
    deg = deg_ref[...]
    dinvp = lax.rsqrt(deg[0] + 1.0)
    dinvn = lax.rsqrt(deg[1] + 1.0)
    hp = jnp.dot(h, wp_ref[...], preferred_element_type=_f32)
    hn = jnp.dot(h, wn_ref[...], preferred_element_type=_f32)
    sp = hp * dinvp
    sn = hn * dinvn
    sp_ref[...] = jnp.stack([sp[:, :32], sp[:, 32:]], axis=0)
    sn_ref[...] = jnp.stack([sn[:, :32], sn[:, 32:]], axis=0)
    selfp_ref[...] = dinvp * dinvp * hp + bp_ref[...]
    selfn_ref[...] = dinvn * dinvn * hn + bn_ref[...]
    dinvp_ref[...] = dinvp
    dinvn_ref[...] = dinvn


def _pre(x, W_enc, b_enc, W_pos, b_pos, W_neg, b_neg, deg):
    n_blk = _N // _BN
    return pl.pallas_call(
        _pre_body,
        grid=(n_blk,),
        in_specs=[
            pl.BlockSpec((_BN, _F), lambda i: (i, 0)),
            pl.BlockSpec((_F, _H), lambda i: (0, 0)),
            pl.BlockSpec((1, _H), lambda i: (0, 0)),
            pl.BlockSpec((_H, _H), lambda i: (0, 0)),
            pl.BlockSpec((1, _H), lambda i: (0, 0)),
            pl.BlockSpec((_H, _H), lambda i: (0, 0)),
            pl.BlockSpec((1, _H), lambda i: (0, 0)),
            pl.BlockSpec((2, _BN, 1), lambda i: (0, i, 0)),
        ],
        out_specs=[
            pl.BlockSpec((2, _BN, 32), lambda i: (0, i, 0)),
            pl.BlockSpec((2, _BN, 32), lambda i: (0, i, 0)),
            pl.BlockSpec((_BN, _H), lambda i: (i, 0)),
            pl.BlockSpec((_BN, _H), lambda i: (i, 0)),
            pl.BlockSpec((_BN, 1), lambda i: (i, 0)),
            pl.BlockSpec((_BN, 1), lambda i: (i, 0)),
        ],
        out_shape=[
            jax.ShapeDtypeStruct((2, _N, 32), _f32),
            jax.ShapeDtypeStruct((2, _N, 32), _f32),
            jax.ShapeDtypeStruct((_N, _H), _f32),
            jax.ShapeDtypeStruct((_N, _H), _f32),
            jax.ShapeDtypeStruct((_N, 1), _f32),
            jax.ShapeDtypeStruct((_N, 1), _f32),
        ],
    )(x, W_enc, b_enc.reshape(1, _H), W_pos, b_pos.reshape(1, _H),
      W_neg, b_neg.reshape(1, _H), deg.reshape(_NC, _NPAD, 1))


def _ode_body(accp_ref, accn_ref, selfp_ref, selfn_ref, dinvp_ref, dinvn_ref,
              w_ref, b_ref, dt_ref, out_ref):
    accp = accp_ref[...]
    accn = accn_ref[...]
    hp = dinvp_ref[...] * jnp.concatenate([accp[0], accp[1]], axis=1) + selfp_ref[...]
    hn = dinvn_ref[...] * jnp.concatenate([accn[0], accn[1]], axis=1) + selfn_ref[...]
    y = jnp.concatenate([hp, hn], axis=1)
    w = w_ref[...]
    b = b_ref[...]
    dt = dt_ref[0, 0]

    def f(z):
        return jnp.tanh(jnp.dot(z, w, preferred_element_type=_f32) + b)

    for _ in range(_STEPS):
        ks = []
        for arow in _A:
            yi = y
            for aij, kj in zip(arow, ks):
                if aij != 0.0:
                    yi = yi + (dt * aij) * kj
            ks.append(f(yi))
        incr = None
        for bi, ki in zip(_B, ks):
            if bi != 0.0:
                incr = (dt * bi) * ki if incr is None else incr + (dt * bi) * ki
        y = y + incr
    out_ref[...] = y


def _ode(acc_p, acc_n, self_p, self_n, dinv_p, dinv_n, W_ode, b_ode, dt):
    n_blk = _N // _BN
    return pl.pallas_call(
        _ode_body,
        grid=(n_blk,),
        in_specs=[
            pl.BlockSpec((2, _BN, 32), lambda i: (0, i, 0)),
            pl.BlockSpec((2, _BN, 32), lambda i: (0, i, 0)),
            pl.BlockSpec((_BN, _H), lambda i: (i, 0)),
            pl.BlockSpec((_BN, _H), lambda i: (i, 0)),
            pl.BlockSpec((_BN, 1), lambda i: (i, 0)),
            pl.BlockSpec((_BN, 1), lambda i: (i, 0)),
            pl.BlockSpec((2 * _H, 2 * _H), lambda i: (0, 0)),
            pl.BlockSpec((1, 2 * _H), lambda i: (0, 0)),
            pl.BlockSpec(memory_space=pltpu.SMEM),
        ],
        out_specs=pl.BlockSpec((_BN, 2 * _H), lambda i: (i, 0)),
        out_shape=jax.ShapeDtypeStruct((_N, 2 * _H), _f32),
    )(acc_p, acc_n, self_p, self_n, dinv_p, dinv_n,
      W_ode, b_ode.reshape(1, 2 * _H), dt)


def kernel(x, pos_edge_index, neg_edge_index, t,
           W_enc, b_enc, W_pos, b_pos, W_neg, b_neg, W_ode, b_ode):
    pad = _EP - _E
    zpad = jnp.zeros((pad,), jnp.int32)
    tpad = jnp.full((pad,), _N, jnp.int32)

    def prep_src(src):
        s2 = jnp.concatenate([src, zpad]).reshape(_EROWS, _LANE)
        return jnp.stack([s2, s2 + _N])

    def prep_dst(dst):
        return jnp.concatenate([dst, tpad]).reshape(_EROWS, _LANE)

    src_p = prep_src(pos_edge_index[0])
    dst_p = prep_dst(pos_edge_index[1])
    src_n = prep_src(neg_edge_index[0])
    dst_n = prep_dst(neg_edge_index[1])

    deg = _deg_kernel()(jnp.stack([dst_p, dst_n]))

    s_p, s_n, self_p, self_n, dinv_p, dinv_n = _pre(
        x, W_enc, b_enc, W_pos, b_pos, W_neg, b_neg, deg)

    acc_p = _scatter_kernel()(s_p.reshape(_NC * _N, 32), src_p, dst_p)
    acc_n = _scatter_kernel()(s_n.reshape(_NC * _N, 32), src_n, dst_n)

    dt = ((t[1] - t[0]) / _STEPS).reshape(1, 1)
    return _ode(acc_p, acc_n, self_p, self_n, dinv_p, dinv_n, W_ode, b_ode, dt)
